# baseline (speedup 1.0000x reference)
import functools

import jax
import jax.numpy as jnp
from jax import lax
from jax.experimental import pallas as pl
from jax.experimental.pallas import tpu as pltpu
from jax.experimental.pallas import tpu_sc as plsc

NC = 2
NS = 16
NW = NC * NS
CHUNK = 128


def _make_gather_kernel(Vp, nch):
    mesh = plsc.VectorSubcoreMesh(core_axis_name="c", subcore_axis_name="s")

    @functools.partial(
        pl.kernel,
        mesh=mesh,
        out_type=jax.ShapeDtypeStruct((NW, nch, CHUNK, 128), jnp.float32),
        scratch_types=[
            pltpu.VMEM((nch, CHUNK), jnp.int32),
            pltpu.VMEM((nch, CHUNK, 128), jnp.float32),
            pltpu.SemaphoreType.DMA,
        ],
    )
    def gather_kernel(pidx_hbm, table_hbm, out_hbm, idx_v, rows_v, sem):
        wid = lax.axis_index("s") * NC + lax.axis_index("c")
        pltpu.sync_copy(pidx_hbm.at[wid], idx_v)
        copies = [
            pltpu.async_copy(table_hbm.at[idx_v.at[j]], rows_v.at[j], sem)
            for j in range(nch)
        ]
        for c in copies:
            c.wait()
        pltpu.sync_copy(rows_v, out_hbm.at[wid])

    return gather_kernel


def kernel(labels, emb_weight):
    (B,) = labels.shape
    V, D = emb_weight.shape
    b_per_w = B // NW
    nch = b_per_w // CHUNK
    lab = labels.astype(jnp.int32)
    pidx = (lab >> 1).reshape(NW, nch, CHUNK)
    table2 = emb_weight.reshape(V // 2, 2 * D)
    pairs = _make_gather_kernel(V // 2, nch)(pidx, table2).reshape(B, 2 * D)
    return jnp.where((lab & 1)[:, None] == 1, pairs[:, D:], pairs[:, :D])


# speedup vs baseline: 1.7189x; 1.7189x over previous
import functools

import jax
import jax.numpy as jnp
from jax import lax
from jax.experimental import pallas as pl
from jax.experimental.pallas import tpu as pltpu
from jax.experimental.pallas import tpu_sc as plsc

NC = 2
NS = 16
NW = NC * NS
L = 16


def _make_gather_kernel(V, D, b_per_w):
    mesh = plsc.VectorSubcoreMesh(core_axis_name="c", subcore_axis_name="s")
    n_g = b_per_w // L

    @functools.partial(
        pl.kernel,
        mesh=mesh,
        out_type=jax.ShapeDtypeStruct((NW, b_per_w, D), jnp.float32),
        scratch_types=[
            pltpu.VMEM((b_per_w,), jnp.int32),
            pltpu.VMEM((b_per_w, D), jnp.float32),
            pltpu.SemaphoreType.DMA,
        ],
    )
    def gather_kernel(lab_hbm, table_hbm, out_hbm, lab_v, rows_v, sem):
        wid = lax.axis_index("s") * NC + lax.axis_index("c")
        pltpu.sync_copy(lab_hbm.at[wid], lab_v)

        def step(g, _):
            lvec = lab_v[pl.ds(g * L, L)]
            for t in range(L):
                pltpu.async_copy(
                    table_hbm.at[lvec[t]], rows_v.at[g * L + t], sem
                )
            @pl.when(g > 0)
            def _():
                pltpu.make_async_copy(
                    table_hbm.at[pl.ds(0, L)],
                    rows_v.at[pl.ds((g - 1) * L, L)],
                    sem,
                ).wait()
            return 0

        lax.fori_loop(0, n_g, step, 0)
        pltpu.make_async_copy(
            table_hbm.at[pl.ds(0, L)],
            rows_v.at[pl.ds((n_g - 1) * L, L)],
            sem,
        ).wait()
        pltpu.sync_copy(rows_v, out_hbm.at[wid])

    return gather_kernel


def kernel(labels, emb_weight):
    (B,) = labels.shape
    V, D = emb_weight.shape
    b_per_w = B // NW
    lab = labels.astype(jnp.int32).reshape(NW, b_per_w)
    out = _make_gather_kernel(V, D, b_per_w)(lab, emb_weight)
    return out.reshape(B, D)
